# baseline (speedup 1.0000x reference)
import functools

import jax
import jax.numpy as jnp
from jax import lax
from jax.experimental import pallas as pl
from jax.experimental.pallas import tpu as pltpu
from jax.experimental.pallas import tpu_sc as plsc

N_HEADS, HEAD_DIM, ROPE_DIM, TOPK = 64, 128, 64, 1024
HEAD_CHUNK = 8
SBLK = 256
TBLK = 256
SCALE = HEAD_DIM ** -0.5
NEG = -1e9


def _fwht(x):
    d = x.shape[-1]
    shp = x.shape
    x = x.reshape(-1, d)
    h = 1
    while h < d:
        x = x.reshape(-1, d // (2 * h), 2, h)
        a = x[:, :, 0, :]
        b = x[:, :, 1, :]
        x = jnp.stack([a + b, a - b], axis=2).reshape(-1, d)
        h *= 2
    return (x * (d ** -0.5)).reshape(shp)


def _apply_rope(x, cos, sin):
    d = x.shape[-1]
    xr = x[..., : d // 2]
    xi = x[..., d // 2 :]
    c = cos.reshape(1, cos.shape[0], 1, cos.shape[1])
    s = sin.reshape(1, sin.shape[0], 1, sin.shape[1])
    return jnp.concatenate([xr * c - xi * s, xr * s + xi * c], axis=-1)


def _layer_norm(x, w, b, eps=1e-6):
    mu = jnp.mean(x, axis=-1, keepdims=True)
    var = jnp.mean((x - mu) ** 2, axis=-1, keepdims=True)
    return (x - mu) / jnp.sqrt(var + eps) * w + b


def _score_kernel(qf_ref, k_ref, w_ref, out_ref):
    i = pl.program_id(0)
    j = pl.program_id(1)

    @pl.when(j > i)
    def _skip():
        out_ref[...] = jnp.full((SBLK, TBLK), NEG, jnp.float32)

    @pl.when(j <= i)
    def _compute():
        kb = k_ref[...]
        score = None
        for c in range(N_HEADS // HEAD_CHUNK):
            ch = None
            for hh in range(HEAD_CHUNK):
                h = c * HEAD_CHUNK + hh
                qh = qf_ref[:, h * HEAD_DIM : (h + 1) * HEAD_DIM]
                sc = jax.lax.dot_general(
                    qh, kb, (((1,), (1,)), ((), ())),
                    preferred_element_type=jnp.float32)
                sc = jnp.maximum(sc, 0.0) * SCALE
                scq = sc.astype(jnp.bfloat16).astype(jnp.float32)
                wq = w_ref[:, h : h + 1].astype(jnp.bfloat16).astype(jnp.float32)
                t = wq * scq
                ch = t if ch is None else ch + t
            score = ch if score is None else score + ch

        @pl.when(j == i)
        def _mask():
            rows = jax.lax.broadcasted_iota(jnp.int32, (SBLK, TBLK), 0)
            cols = jax.lax.broadcasted_iota(jnp.int32, (SBLK, TBLK), 1)
            out_ref[...] = score + jnp.where(cols <= rows, 0.0, NEG).astype(jnp.float32)

        @pl.when(j < i)
        def _nomask():
            out_ref[...] = score


def _scores(qf, k, w, s):
    return pl.pallas_call(
        _score_kernel,
        grid=(s // SBLK, s // TBLK),
        in_specs=[
            pl.BlockSpec((SBLK, N_HEADS * HEAD_DIM), lambda i, j: (i, 0)),
            pl.BlockSpec((TBLK, HEAD_DIM), lambda i, j: (j, 0)),
            pl.BlockSpec((SBLK, N_HEADS), lambda i, j: (i, 0)),
        ],
        out_specs=pl.BlockSpec((SBLK, TBLK), lambda i, j: (i, j)),
        out_shape=jax.ShapeDtypeStruct((s, s), jnp.float32),
    )(qf, k, w)


def _topk_sc(masked, s):
    NW = 32
    rows_per = s // NW
    mesh = plsc.VectorSubcoreMesh(core_axis_name="c", subcore_axis_name="s")

    @functools.partial(
        pl.kernel,
        mesh=mesh,
        compiler_params=pltpu.CompilerParams(needs_layout_passes=False),
        out_type=[
            jax.ShapeDtypeStruct((s, TOPK), jnp.float32),
            jax.ShapeDtypeStruct((s, TOPK), jnp.int32),
        ],
        scratch_types=[
            pltpu.VMEM((s,), jnp.float32),
            pltpu.VMEM((s,), jnp.int32),
            pltpu.VMEM((s,), jnp.int32),
            pltpu.VMEM((s,), jnp.int32),
            pltpu.VMEM((s,), jnp.int32),
            pltpu.VMEM((512,), jnp.int32),
            pltpu.VMEM((32,), jnp.int32),
            pltpu.VMEM((16,), jnp.int32),
            pltpu.VMEM((TOPK,), jnp.float32),
            pltpu.VMEM((TOPK,), jnp.int32),
        ],
    )
    def tk(scores_hbm, vals_hbm, idx_hbm, row_f, keyA, valA, keyB, valB,
           hist, cnt, tmp16, ovals, oidx):
        wid = lax.axis_index("s") * 2 + lax.axis_index("c")
        lane = lax.iota(jnp.int32, 16)
        ones16 = jnp.ones((16,), jnp.int32)

        def row_body(rr, _):
            row = rr * NW + wid
            pltpu.sync_copy(scores_hbm.at[row], row_f)
            nv = row // 16 + 1

            def kv_body(i, _c):
                off = pl.multiple_of(i * 16, 16)
                f = row_f[pl.ds(off, 16)]
                u = lax.bitcast_convert_type(f, jnp.int32)
                m = jnp.bitwise_not(lax.shift_right_arithmetic(u, 31)) & 0x7FFFFFFF
                keyA[pl.ds(off, 16)] = u ^ m
                valA[pl.ds(off, 16)] = lane + i * 16
                return 0

            lax.fori_loop(0, nv, kv_body, 0)

            bufs = [(keyA, valA, keyB, valB), (keyB, valB, keyA, valA)]
            for p in range(7):
                ksrc, vsrc, kdst, vdst = bufs[p % 2]
                shift = 5 * p

                def clr(i, _c):
                    hist[pl.ds(pl.multiple_of(i * 16, 16), 16)] = jnp.zeros((16,), jnp.int32)
                    return 0

                lax.fori_loop(0, 32, clr, 0)

                def hist_body(i, _c, _ksrc=ksrc, _shift=shift):
                    kk = _ksrc[pl.ds(pl.multiple_of(i * 16, 16), 16)]
                    dd = lax.shift_right_logical(kk, _shift) & 31
                    slot = dd * 16 + lane
                    h0 = plsc.load_gather(hist, [slot])
                    plsc.store_scatter(hist, [slot], h0 + ones16)
                    return 0

                lax.fori_loop(0, nv, hist_body, 0)

                tot0 = jnp.zeros((16,), jnp.int32)
                tot1 = jnp.zeros((16,), jnp.int32)
                for l in range(16):
                    tot0 = tot0 + plsc.load_gather(hist, [lane * 16 + l])
                    tot1 = tot1 + plsc.load_gather(hist, [(lane + 16) * 16 + l])
                cs0 = plsc.cumsum(tot0)
                cs1 = plsc.cumsum(tot1)
                t0sum = jnp.sum(tot0)
                cnt[pl.ds(0, 16)] = cs0 - tot0
                cnt[pl.ds(16, 16)] = cs1 - tot1 + t0sum

                def perm(i, _c, _ksrc=ksrc, _vsrc=vsrc, _kdst=kdst, _vdst=vdst, _shift=shift):
                    off = pl.multiple_of(i * 16, 16)
                    kk = _ksrc[pl.ds(off, 16)]
                    dd = lax.shift_right_logical(kk, _shift) & 31
                    sdl, slane = plsc.sort_key_val(dd * 16 + lane, lane)
                    sd = lax.shift_right_logical(sdl, 4)
                    tmp16[...] = sd
                    prev = plsc.load_gather(tmp16, [jnp.maximum(lane - 1, 0)])
                    nxt = plsc.load_gather(tmp16, [jnp.minimum(lane + 1, 15)])
                    is_start = (lane == 0) | (prev != sd)
                    is_end = (lane == 15) | (nxt != sd)
                    run_start = plsc.cummax(jnp.where(is_start, lane, 0))
                    rank = lane - run_start
                    base = plsc.load_gather(cnt, [sd])
                    pos = base + rank
                    gidx = i * 16 + slane
                    sk = plsc.load_gather(_ksrc, [gidx])
                    sv = plsc.load_gather(_vsrc, [gidx])
                    plsc.store_scatter(_kdst, [pos], sk)
                    plsc.store_scatter(_vdst, [pos], sv)
                    plsc.store_scatter(cnt, [sd], base + rank + 1, mask=is_end)
                    return 0

                lax.fori_loop(0, nv, perm, 0)

            nout = jnp.minimum(nv, TOPK // 16)

            def out_body(j, _c):
                off = pl.multiple_of(j * 16, 16)
                iv = valB[pl.ds(off, 16)]
                ovals[pl.ds(off, 16)] = plsc.load_gather(row_f, [iv])
                oidx[pl.ds(off, 16)] = iv
                return 0

            lax.fori_loop(0, nout, out_body, 0)

            def fill_body(j, _c):
                off = pl.multiple_of(j * 16, 16)
                oidx[pl.ds(off, 16)] = lane + j * 16
                ovals[pl.ds(off, 16)] = jnp.full((16,), NEG, jnp.float32)
                return 0

            lax.fori_loop(nout, TOPK // 16, fill_body, 0)

            pltpu.sync_copy(ovals, vals_hbm.at[row])
            pltpu.sync_copy(oidx, idx_hbm.at[row])
            return 0

        lax.fori_loop(0, rows_per, row_body, 0)

    return tk(masked)


def kernel(hidden_states, q_lora, freqs_cos, freqs_sin, wq_b, wk, k_norm_w, k_norm_b, w_proj):
    b, s, d = hidden_states.shape
    q = (q_lora @ wq_b).reshape(b, s, N_HEADS, HEAD_DIM)
    q_pe = _apply_rope(q[..., :ROPE_DIM], freqs_cos, freqs_sin)
    q = jnp.concatenate([q_pe, q[..., ROPE_DIM:]], axis=-1)
    k = hidden_states @ wk
    k = _layer_norm(k, k_norm_w, k_norm_b)
    k_pe = _apply_rope(k[..., :ROPE_DIM][:, :, None, :], freqs_cos, freqs_sin)[:, :, 0, :]
    k = jnp.concatenate([k_pe, k[..., ROPE_DIM:]], axis=-1)
    q = _fwht(q)
    k = _fwht(k)
    weights = (hidden_states @ w_proj) * (N_HEADS ** -0.5)

    qf = q.reshape(s, N_HEADS * HEAD_DIM)
    masked = _scores(qf, k[0], weights[0], s)

    topk_vals, topk_idx = _topk_sc(masked, s)
    return topk_vals[None], topk_idx[None]


# speedup vs baseline: 1.6294x; 1.6294x over previous
import functools

import jax
import jax.numpy as jnp
from jax import lax
from jax.experimental import pallas as pl
from jax.experimental.pallas import tpu as pltpu
from jax.experimental.pallas import tpu_sc as plsc

N_HEADS, HEAD_DIM, ROPE_DIM, TOPK = 64, 128, 64, 1024
Q_LORA_RANK = 1536
HEAD_CHUNK = 8
SBLK = 256
TBLK = 256
SCALE = HEAD_DIM ** -0.5
NEG = -1e9


def _fwht(x):
    d = x.shape[-1]
    shp = x.shape
    x = x.reshape(-1, d)
    h = 1
    while h < d:
        x = x.reshape(-1, d // (2 * h), 2, h)
        a = x[:, :, 0, :]
        b = x[:, :, 1, :]
        x = jnp.stack([a + b, a - b], axis=2).reshape(-1, d)
        h *= 2
    return (x * (d ** -0.5)).reshape(shp)


def _apply_rope(x, cos, sin):
    d = x.shape[-1]
    xr = x[..., : d // 2]
    xi = x[..., d // 2 :]
    c = cos.reshape(1, cos.shape[0], 1, cos.shape[1])
    s = sin.reshape(1, sin.shape[0], 1, sin.shape[1])
    return jnp.concatenate([xr * c - xi * s, xr * s + xi * c], axis=-1)


def _layer_norm(x, w, b, eps=1e-6):
    mu = jnp.mean(x, axis=-1, keepdims=True)
    var = jnp.mean((x - mu) ** 2, axis=-1, keepdims=True)
    return (x - mu) / jnp.sqrt(var + eps) * w + b


def _score_kernel(qf_ref, k_ref, w_ref, out_ref):
    i = pl.program_id(0)
    j = pl.program_id(1)

    @pl.when(j > i)
    def _skip():
        out_ref[...] = jnp.full((SBLK, TBLK), NEG, jnp.float32)

    @pl.when(j <= i)
    def _compute():
        kb = k_ref[...]
        score = None
        for c in range(N_HEADS // HEAD_CHUNK):
            ch = None
            for hh in range(HEAD_CHUNK):
                h = c * HEAD_CHUNK + hh
                qh = qf_ref[:, h * HEAD_DIM : (h + 1) * HEAD_DIM]
                sc = jax.lax.dot_general(
                    qh, kb, (((1,), (1,)), ((), ())),
                    preferred_element_type=jnp.float32)
                sc = jnp.maximum(sc, 0.0) * SCALE
                scq = sc.astype(jnp.bfloat16).astype(jnp.float32)
                wq = w_ref[:, h : h + 1].astype(jnp.bfloat16).astype(jnp.float32)
                t = wq * scq
                ch = t if ch is None else ch + t
            score = ch if score is None else score + ch

        @pl.when(j == i)
        def _mask():
            rows = jax.lax.broadcasted_iota(jnp.int32, (SBLK, TBLK), 0)
            cols = jax.lax.broadcasted_iota(jnp.int32, (SBLK, TBLK), 1)
            out_ref[...] = score + jnp.where(cols <= rows, 0.0, NEG).astype(jnp.float32)

        @pl.when(j < i)
        def _nomask():
            out_ref[...] = score


def _scores(qf, k, w, s):
    return pl.pallas_call(
        _score_kernel,
        grid=(s // SBLK, s // TBLK),
        in_specs=[
            pl.BlockSpec((SBLK, N_HEADS * HEAD_DIM), lambda i, j: (i, 0)),
            pl.BlockSpec((TBLK, HEAD_DIM), lambda i, j: (j, 0)),
            pl.BlockSpec((SBLK, N_HEADS), lambda i, j: (i, 0)),
        ],
        out_specs=pl.BlockSpec((SBLK, TBLK), lambda i, j: (i, j)),
        out_shape=jax.ShapeDtypeStruct((s, s), jnp.float32),
    )(qf, k, w)


def _q_kernel(ql_ref, wq_ref, cos_ref, sin_ref, out_ref):
    x = jnp.dot(ql_ref[...], wq_ref[...], preferred_element_type=jnp.float32)
    lane128 = jax.lax.broadcasted_iota(jnp.int32, (SBLK, HEAD_DIM), 1)
    c = cos_ref[...]
    s_ = sin_ref[...]
    fw_scale = HEAD_DIM ** -0.5
    outs = []
    for hh in range(8):
        xh = x[:, hh * HEAD_DIM : (hh + 1) * HEAD_DIM]
        xr = xh[:, : ROPE_DIM // 2]
        xi = xh[:, ROPE_DIM // 2 : ROPE_DIM]
        rot = jnp.concatenate([xr * c - xi * s_, xr * s_ + xi * c], axis=1)
        y = jnp.concatenate([rot, xh[:, ROPE_DIM:]], axis=1)
        for st in range(7):
            hd = 1 << st
            left = jnp.concatenate([y[:, hd:], y[:, :hd]], axis=1)
            right = jnp.concatenate([y[:, HEAD_DIM - hd :], y[:, : HEAD_DIM - hd]], axis=1)
            y = jnp.where((lane128 & hd) == 0, y + left, right - y)
        outs.append(y * fw_scale)
    out_ref[...] = jnp.concatenate(outs, axis=1)


def _q_proj(ql, wq_b, cos, sin, s):
    nblk = 1024
    return pl.pallas_call(
        _q_kernel,
        grid=(N_HEADS * HEAD_DIM // nblk, s // SBLK),
        in_specs=[
            pl.BlockSpec((SBLK, Q_LORA_RANK), lambda n, i: (i, 0)),
            pl.BlockSpec((Q_LORA_RANK, nblk), lambda n, i: (0, n)),
            pl.BlockSpec((SBLK, ROPE_DIM // 2), lambda n, i: (i, 0)),
            pl.BlockSpec((SBLK, ROPE_DIM // 2), lambda n, i: (i, 0)),
        ],
        out_specs=pl.BlockSpec((SBLK, nblk), lambda n, i: (i, n)),
        out_shape=jax.ShapeDtypeStruct((s, N_HEADS * HEAD_DIM), jnp.float32),
    )(ql, wq_b, cos, sin)


def _topk_sc(masked, s):
    NW = 32
    rows_per = s // NW
    mesh = plsc.VectorSubcoreMesh(core_axis_name="c", subcore_axis_name="s")

    @functools.partial(
        pl.kernel,
        mesh=mesh,
        compiler_params=pltpu.CompilerParams(needs_layout_passes=False),
        out_type=[
            jax.ShapeDtypeStruct((s, TOPK), jnp.float32),
            jax.ShapeDtypeStruct((s, TOPK), jnp.int32),
        ],
        scratch_types=[
            pltpu.VMEM((s,), jnp.float32),
            pltpu.VMEM((s,), jnp.int32),
            pltpu.VMEM((s,), jnp.int32),
            pltpu.VMEM((s,), jnp.int32),
            pltpu.VMEM((s,), jnp.int32),
            pltpu.VMEM((512,), jnp.int32),
            pltpu.VMEM((32,), jnp.int32),
            pltpu.VMEM((16,), jnp.int32),
            pltpu.VMEM((TOPK,), jnp.float32),
            pltpu.VMEM((TOPK,), jnp.int32),
        ],
    )
    def tk(scores_hbm, vals_hbm, idx_hbm, row_f, keyA, valA, keyB, valB,
           hist, cnt, tmp16, ovals, oidx):
        wid = lax.axis_index("s") * 2 + lax.axis_index("c")
        lane = lax.iota(jnp.int32, 16)
        ones16 = jnp.ones((16,), jnp.int32)

        def row_body(rr, _):
            row = rr * NW + wid
            pltpu.sync_copy(scores_hbm.at[row], row_f)
            nv = row // 16 + 1

            def kv_body(i, _c):
                off = pl.multiple_of(i * 16, 16)
                f = row_f[pl.ds(off, 16)]
                u = lax.bitcast_convert_type(f, jnp.int32)
                m = jnp.bitwise_not(lax.shift_right_arithmetic(u, 31)) & 0x7FFFFFFF
                keyA[pl.ds(off, 16)] = u ^ m
                valA[pl.ds(off, 16)] = lane + i * 16
                return 0

            lax.fori_loop(0, nv, kv_body, 0)

            bufs = [(keyA, valA, keyB, valB), (keyB, valB, keyA, valA)]
            for p in range(7):
                ksrc, vsrc, kdst, vdst = bufs[p % 2]
                shift = 5 * p

                def clr(i, _c):
                    hist[pl.ds(pl.multiple_of(i * 16, 16), 16)] = jnp.zeros((16,), jnp.int32)
                    return 0

                lax.fori_loop(0, 32, clr, 0)

                def hist_body(i, _c, _ksrc=ksrc, _shift=shift):
                    kk = _ksrc[pl.ds(pl.multiple_of(i * 16, 16), 16)]
                    dd = lax.shift_right_logical(kk, _shift) & 31
                    slot = dd * 16 + lane
                    h0 = plsc.load_gather(hist, [slot])
                    plsc.store_scatter(hist, [slot], h0 + ones16)
                    return 0

                lax.fori_loop(0, nv, hist_body, 0)

                tot0 = jnp.zeros((16,), jnp.int32)
                tot1 = jnp.zeros((16,), jnp.int32)
                for l in range(16):
                    tot0 = tot0 + plsc.load_gather(hist, [lane * 16 + l])
                    tot1 = tot1 + plsc.load_gather(hist, [(lane + 16) * 16 + l])
                cs0 = plsc.cumsum(tot0)
                cs1 = plsc.cumsum(tot1)
                t0sum = jnp.sum(tot0)
                cnt[pl.ds(0, 16)] = cs0 - tot0
                cnt[pl.ds(16, 16)] = cs1 - tot1 + t0sum

                def perm(i, _c, _ksrc=ksrc, _vsrc=vsrc, _kdst=kdst, _vdst=vdst, _shift=shift):
                    off = pl.multiple_of(i * 16, 16)
                    kk = _ksrc[pl.ds(off, 16)]
                    dd = lax.shift_right_logical(kk, _shift) & 31
                    sdl, slane = plsc.sort_key_val(dd * 16 + lane, lane)
                    sd = lax.shift_right_logical(sdl, 4)
                    tmp16[...] = sd
                    prev = plsc.load_gather(tmp16, [jnp.maximum(lane - 1, 0)])
                    nxt = plsc.load_gather(tmp16, [jnp.minimum(lane + 1, 15)])
                    is_start = (lane == 0) | (prev != sd)
                    is_end = (lane == 15) | (nxt != sd)
                    run_start = plsc.cummax(jnp.where(is_start, lane, 0))
                    rank = lane - run_start
                    base = plsc.load_gather(cnt, [sd])
                    pos = base + rank
                    gidx = i * 16 + slane
                    sk = plsc.load_gather(_ksrc, [gidx])
                    sv = plsc.load_gather(_vsrc, [gidx])
                    plsc.store_scatter(_kdst, [pos], sk)
                    plsc.store_scatter(_vdst, [pos], sv)
                    plsc.store_scatter(cnt, [sd], base + rank + 1, mask=is_end)
                    return 0

                lax.fori_loop(0, nv, perm, 0)

            nout = jnp.minimum(nv, TOPK // 16)

            def out_body(j, _c):
                off = pl.multiple_of(j * 16, 16)
                iv = valB[pl.ds(off, 16)]
                ovals[pl.ds(off, 16)] = plsc.load_gather(row_f, [iv])
                oidx[pl.ds(off, 16)] = iv
                return 0

            lax.fori_loop(0, nout, out_body, 0)

            def fill_body(j, _c):
                off = pl.multiple_of(j * 16, 16)
                oidx[pl.ds(off, 16)] = lane + j * 16
                ovals[pl.ds(off, 16)] = jnp.full((16,), NEG, jnp.float32)
                return 0

            lax.fori_loop(nout, TOPK // 16, fill_body, 0)

            pltpu.sync_copy(ovals, vals_hbm.at[row])
            pltpu.sync_copy(oidx, idx_hbm.at[row])
            return 0

        lax.fori_loop(0, rows_per, row_body, 0)

    return tk(masked)


def kernel(hidden_states, q_lora, freqs_cos, freqs_sin, wq_b, wk, k_norm_w, k_norm_b, w_proj):
    b, s, d = hidden_states.shape
    k = hidden_states @ wk
    k = _layer_norm(k, k_norm_w, k_norm_b)
    k_pe = _apply_rope(k[..., :ROPE_DIM][:, :, None, :], freqs_cos, freqs_sin)[:, :, 0, :]
    k = jnp.concatenate([k_pe, k[..., ROPE_DIM:]], axis=-1)
    k = _fwht(k)
    weights = (hidden_states @ w_proj) * (N_HEADS ** -0.5)

    qf = _q_proj(q_lora[0], wq_b, freqs_cos, freqs_sin, s)
    masked = _scores(qf, k[0], weights[0], s)

    topk_vals, topk_idx = _topk_sc(masked, s)
    return topk_vals[None], topk_idx[None]


# speedup vs baseline: 2.0904x; 1.2829x over previous
import functools

import jax
import jax.numpy as jnp
from jax import lax
from jax.experimental import pallas as pl
from jax.experimental.pallas import tpu as pltpu
from jax.experimental.pallas import tpu_sc as plsc

N_HEADS, HEAD_DIM, ROPE_DIM, TOPK = 64, 128, 64, 1024
Q_LORA_RANK = 1536
HEAD_CHUNK = 8
SBLK = 256
TBLK = 256
SCALE = HEAD_DIM ** -0.5
NEG = -1e9


def _fwht(x):
    d = x.shape[-1]
    shp = x.shape
    x = x.reshape(-1, d)
    h = 1
    while h < d:
        x = x.reshape(-1, d // (2 * h), 2, h)
        a = x[:, :, 0, :]
        b = x[:, :, 1, :]
        x = jnp.stack([a + b, a - b], axis=2).reshape(-1, d)
        h *= 2
    return (x * (d ** -0.5)).reshape(shp)


def _apply_rope(x, cos, sin):
    d = x.shape[-1]
    xr = x[..., : d // 2]
    xi = x[..., d // 2 :]
    c = cos.reshape(1, cos.shape[0], 1, cos.shape[1])
    s = sin.reshape(1, sin.shape[0], 1, sin.shape[1])
    return jnp.concatenate([xr * c - xi * s, xr * s + xi * c], axis=-1)


def _layer_norm(x, w, b, eps=1e-6):
    mu = jnp.mean(x, axis=-1, keepdims=True)
    var = jnp.mean((x - mu) ** 2, axis=-1, keepdims=True)
    return (x - mu) / jnp.sqrt(var + eps) * w + b


def _score_kernel(qf_ref, k_ref, w_ref, out_ref):
    i = pl.program_id(0)
    j = pl.program_id(1)

    @pl.when(j > i)
    def _skip():
        out_ref[...] = jnp.full((SBLK, TBLK), NEG, jnp.float32)

    @pl.when(j <= i)
    def _compute():
        kb = k_ref[...]
        score = None
        for c in range(N_HEADS // HEAD_CHUNK):
            ch = None
            for hh in range(HEAD_CHUNK):
                h = c * HEAD_CHUNK + hh
                qh = qf_ref[:, h * HEAD_DIM : (h + 1) * HEAD_DIM]
                sc = jax.lax.dot_general(
                    qh, kb, (((1,), (1,)), ((), ())),
                    preferred_element_type=jnp.float32)
                sc = jnp.maximum(sc, 0.0) * SCALE
                scq = sc.astype(jnp.bfloat16).astype(jnp.float32)
                wq = w_ref[:, h : h + 1].astype(jnp.bfloat16).astype(jnp.float32)
                t = wq * scq
                ch = t if ch is None else ch + t
            score = ch if score is None else score + ch

        @pl.when(j == i)
        def _mask():
            rows = jax.lax.broadcasted_iota(jnp.int32, (SBLK, TBLK), 0)
            cols = jax.lax.broadcasted_iota(jnp.int32, (SBLK, TBLK), 1)
            out_ref[...] = score + jnp.where(cols <= rows, 0.0, NEG).astype(jnp.float32)

        @pl.when(j < i)
        def _nomask():
            out_ref[...] = score


def _scores(qf, k, w, s):
    return pl.pallas_call(
        _score_kernel,
        grid=(s // SBLK, s // TBLK),
        in_specs=[
            pl.BlockSpec((SBLK, N_HEADS * HEAD_DIM), lambda i, j: (i, 0)),
            pl.BlockSpec((TBLK, HEAD_DIM), lambda i, j: (j, 0)),
            pl.BlockSpec((SBLK, N_HEADS), lambda i, j: (i, 0)),
        ],
        out_specs=pl.BlockSpec((SBLK, TBLK), lambda i, j: (i, j)),
        out_shape=jax.ShapeDtypeStruct((s, s), jnp.float32),
    )(qf, k, w)


def _q_kernel(ql_ref, wq_ref, cos_ref, sin_ref, out_ref):
    x = jnp.dot(ql_ref[...], wq_ref[...], preferred_element_type=jnp.float32)
    lane128 = jax.lax.broadcasted_iota(jnp.int32, (SBLK, HEAD_DIM), 1)
    c = cos_ref[...]
    s_ = sin_ref[...]
    fw_scale = HEAD_DIM ** -0.5
    outs = []
    for hh in range(8):
        xh = x[:, hh * HEAD_DIM : (hh + 1) * HEAD_DIM]
        xr = xh[:, : ROPE_DIM // 2]
        xi = xh[:, ROPE_DIM // 2 : ROPE_DIM]
        rot = jnp.concatenate([xr * c - xi * s_, xr * s_ + xi * c], axis=1)
        y = jnp.concatenate([rot, xh[:, ROPE_DIM:]], axis=1)
        for st in range(7):
            hd = 1 << st
            left = jnp.concatenate([y[:, hd:], y[:, :hd]], axis=1)
            right = jnp.concatenate([y[:, HEAD_DIM - hd :], y[:, : HEAD_DIM - hd]], axis=1)
            y = jnp.where((lane128 & hd) == 0, y + left, right - y)
        outs.append(y * fw_scale)
    out_ref[...] = jnp.concatenate(outs, axis=1)


def _q_proj(ql, wq_b, cos, sin, s):
    nblk = 1024
    return pl.pallas_call(
        _q_kernel,
        grid=(N_HEADS * HEAD_DIM // nblk, s // SBLK),
        in_specs=[
            pl.BlockSpec((SBLK, Q_LORA_RANK), lambda n, i: (i, 0)),
            pl.BlockSpec((Q_LORA_RANK, nblk), lambda n, i: (0, n)),
            pl.BlockSpec((SBLK, ROPE_DIM // 2), lambda n, i: (i, 0)),
            pl.BlockSpec((SBLK, ROPE_DIM // 2), lambda n, i: (i, 0)),
        ],
        out_specs=pl.BlockSpec((SBLK, nblk), lambda n, i: (i, n)),
        out_shape=jax.ShapeDtypeStruct((s, N_HEADS * HEAD_DIM), jnp.float32),
    )(ql, wq_b, cos, sin)


def _topk_sc(masked, s):
    NW = 32
    NB = 256
    rows_per = s // NW
    mesh = plsc.VectorSubcoreMesh(core_axis_name="c", subcore_axis_name="s")

    @functools.partial(
        pl.kernel,
        mesh=mesh,
        compiler_params=pltpu.CompilerParams(needs_layout_passes=False),
        out_type=[
            jax.ShapeDtypeStruct((s, TOPK), jnp.float32),
            jax.ShapeDtypeStruct((s, TOPK), jnp.int32),
        ],
        scratch_types=[
            pltpu.VMEM((s,), jnp.float32),
            pltpu.VMEM((s,), jnp.int32),
            pltpu.VMEM((s,), jnp.int32),
            pltpu.VMEM((s,), jnp.int32),
            pltpu.VMEM((s,), jnp.int32),
            pltpu.VMEM((NB * 16,), jnp.int32),
            pltpu.VMEM((NB,), jnp.int32),
            pltpu.VMEM((16,), jnp.int32),
            pltpu.VMEM((TOPK,), jnp.float32),
            pltpu.VMEM((TOPK,), jnp.int32),
        ],
    )
    def tk(scores_hbm, vals_hbm, idx_hbm, row_f, keyA, valA, keyB, valB,
           hist, cnt, tmp16, ovals, oidx):
        wid = lax.axis_index("s") * 2 + lax.axis_index("c")
        lane = lax.iota(jnp.int32, 16)
        ones16 = jnp.ones((16,), jnp.int32)

        def hist_rmw(dd):
            slot = dd * 16 + lane
            h0 = plsc.load_gather(hist, [slot])
            plsc.store_scatter(hist, [slot], h0 + ones16)

        def clear_hist():
            def clr(i, _c):
                off = pl.multiple_of(i * 16, 16)
                for u in range(16):
                    hist[pl.ds(off * 16 + u * 16, 16)] = jnp.zeros((16,), jnp.int32)
                return 0
            lax.fori_loop(0, NB // 16, clr, 0)

        def row_body(rr, _):
            row = rr * NW + wid
            pltpu.sync_copy(scores_hbm.at[row], row_f)
            nv = row // 16 + 1

            clear_hist()

            def kv_body(i, _c):
                off = pl.multiple_of(i * 16, 16)
                f = row_f[pl.ds(off, 16)]
                u = lax.bitcast_convert_type(f, jnp.int32)
                m = jnp.bitwise_not(lax.shift_right_arithmetic(u, 31)) & 0x7FFFFFFF
                key = u ^ m
                keyA[pl.ds(off, 16)] = key
                valA[pl.ds(off, 16)] = lane + i * 16
                hist_rmw(key & (NB - 1))
                return 0

            lax.fori_loop(0, nv, kv_body, 0)

            bufs = [(keyA, valA, keyB, valB), (keyB, valB, keyA, valA)]
            for p in range(4):
                ksrc, vsrc, kdst, vdst = bufs[p % 2]
                shift = 8 * p

                def totals(dv, carry):
                    dig = dv * 16 + lane
                    tot = jnp.zeros((16,), jnp.int32)
                    for l in range(16):
                        tot = tot + plsc.load_gather(hist, [dig * 16 + l])
                    cs = plsc.cumsum(tot)
                    cnt[pl.ds(pl.multiple_of(dv * 16, 16), 16)] = cs - tot + carry
                    return carry + jnp.sum(tot)

                lax.fori_loop(0, NB // 16, totals, jnp.int32(0))

                clear_hist()

                def perm(i, _c, _ksrc=ksrc, _vsrc=vsrc, _kdst=kdst, _vdst=vdst,
                         _shift=shift, _last=(p == 3)):
                    off = pl.multiple_of(i * 16, 16)
                    kk = _ksrc[pl.ds(off, 16)]
                    dd = lax.shift_right_logical(kk, _shift) & (NB - 1)
                    sdl, slane = plsc.sort_key_val(dd * 16 + lane, lane)
                    sd = lax.shift_right_logical(sdl, 4)
                    tmp16[...] = sd
                    prev = plsc.load_gather(tmp16, [jnp.maximum(lane - 1, 0)])
                    nxt = plsc.load_gather(tmp16, [jnp.minimum(lane + 1, 15)])
                    is_start = (lane == 0) | (prev != sd)
                    is_end = (lane == 15) | (nxt != sd)
                    run_start = plsc.cummax(jnp.where(is_start, lane, 0))
                    rank = lane - run_start
                    base = plsc.load_gather(cnt, [sd])
                    pos = base + rank
                    gidx = i * 16 + slane
                    sk = plsc.load_gather(_ksrc, [gidx])
                    sv = plsc.load_gather(_vsrc, [gidx])
                    plsc.store_scatter(_kdst, [pos], sk)
                    plsc.store_scatter(_vdst, [pos], sv)
                    plsc.store_scatter(cnt, [sd], base + rank + 1, mask=is_end)
                    if not _last:
                        hist_rmw(lax.shift_right_logical(kk, _shift + 8) & (NB - 1))
                    return 0

                lax.fori_loop(0, nv, perm, 0)

            nout = jnp.minimum(nv, TOPK // 16)

            def out_body(j, _c):
                off = pl.multiple_of(j * 16, 16)
                iv = valA[pl.ds(off, 16)]
                ovals[pl.ds(off, 16)] = plsc.load_gather(row_f, [iv])
                oidx[pl.ds(off, 16)] = iv
                return 0

            lax.fori_loop(0, nout, out_body, 0)

            def fill_body(j, _c):
                off = pl.multiple_of(j * 16, 16)
                oidx[pl.ds(off, 16)] = lane + j * 16
                ovals[pl.ds(off, 16)] = jnp.full((16,), NEG, jnp.float32)
                return 0

            lax.fori_loop(nout, TOPK // 16, fill_body, 0)

            pltpu.sync_copy(ovals, vals_hbm.at[row])
            pltpu.sync_copy(oidx, idx_hbm.at[row])
            return 0

        lax.fori_loop(0, rows_per, row_body, 0)

    return tk(masked)


def kernel(hidden_states, q_lora, freqs_cos, freqs_sin, wq_b, wk, k_norm_w, k_norm_b, w_proj):
    b, s, d = hidden_states.shape
    k = hidden_states @ wk
    k = _layer_norm(k, k_norm_w, k_norm_b)
    k_pe = _apply_rope(k[..., :ROPE_DIM][:, :, None, :], freqs_cos, freqs_sin)[:, :, 0, :]
    k = jnp.concatenate([k_pe, k[..., ROPE_DIM:]], axis=-1)
    k = _fwht(k)
    weights = (hidden_states @ w_proj) * (N_HEADS ** -0.5)

    qf = _q_proj(q_lora[0], wq_b, freqs_cos, freqs_sin, s)
    masked = _scores(qf, k[0], weights[0], s)

    topk_vals, topk_idx = _topk_sc(masked, s)
    return topk_vals[None], topk_idx[None]


# speedup vs baseline: 2.1470x; 1.0271x over previous
import functools

import jax
import jax.numpy as jnp
from jax import lax
from jax.experimental import pallas as pl
from jax.experimental.pallas import tpu as pltpu
from jax.experimental.pallas import tpu_sc as plsc

N_HEADS, HEAD_DIM, ROPE_DIM, TOPK = 64, 128, 64, 1024
Q_LORA_RANK = 1536
HEAD_CHUNK = 8
SBLK = 256
TBLK = 256
SCALE = HEAD_DIM ** -0.5
NEG = -1e9


def _fwht(x):
    d = x.shape[-1]
    shp = x.shape
    x = x.reshape(-1, d)
    h = 1
    while h < d:
        x = x.reshape(-1, d // (2 * h), 2, h)
        a = x[:, :, 0, :]
        b = x[:, :, 1, :]
        x = jnp.stack([a + b, a - b], axis=2).reshape(-1, d)
        h *= 2
    return (x * (d ** -0.5)).reshape(shp)


def _apply_rope(x, cos, sin):
    d = x.shape[-1]
    xr = x[..., : d // 2]
    xi = x[..., d // 2 :]
    c = cos.reshape(1, cos.shape[0], 1, cos.shape[1])
    s = sin.reshape(1, sin.shape[0], 1, sin.shape[1])
    return jnp.concatenate([xr * c - xi * s, xr * s + xi * c], axis=-1)


def _layer_norm(x, w, b, eps=1e-6):
    mu = jnp.mean(x, axis=-1, keepdims=True)
    var = jnp.mean((x - mu) ** 2, axis=-1, keepdims=True)
    return (x - mu) / jnp.sqrt(var + eps) * w + b


def _score_kernel(qf_ref, k_ref, w_ref, out_ref):
    i = pl.program_id(0)
    j = pl.program_id(1)

    @pl.when(j > i)
    def _skip():
        out_ref[...] = jnp.full((SBLK, TBLK), NEG, jnp.float32)

    @pl.when(j <= i)
    def _compute():
        kb = k_ref[...]
        score = None
        for c in range(N_HEADS // HEAD_CHUNK):
            ch = None
            for hh in range(HEAD_CHUNK):
                h = c * HEAD_CHUNK + hh
                qh = qf_ref[:, h * HEAD_DIM : (h + 1) * HEAD_DIM]
                sc = jax.lax.dot_general(
                    qh, kb, (((1,), (1,)), ((), ())),
                    preferred_element_type=jnp.float32)
                sc = jnp.maximum(sc, 0.0) * SCALE
                scq = sc.astype(jnp.bfloat16).astype(jnp.float32)
                wq = w_ref[:, h : h + 1].astype(jnp.bfloat16).astype(jnp.float32)
                t = wq * scq
                ch = t if ch is None else ch + t
            score = ch if score is None else score + ch

        @pl.when(j == i)
        def _mask():
            rows = jax.lax.broadcasted_iota(jnp.int32, (SBLK, TBLK), 0)
            cols = jax.lax.broadcasted_iota(jnp.int32, (SBLK, TBLK), 1)
            out_ref[...] = score + jnp.where(cols <= rows, 0.0, NEG).astype(jnp.float32)

        @pl.when(j < i)
        def _nomask():
            out_ref[...] = score


def _scores(qf, k, w, s):
    return pl.pallas_call(
        _score_kernel,
        grid=(s // SBLK, s // TBLK),
        in_specs=[
            pl.BlockSpec((SBLK, N_HEADS * HEAD_DIM), lambda i, j: (i, 0)),
            pl.BlockSpec((TBLK, HEAD_DIM), lambda i, j: (j, 0)),
            pl.BlockSpec((SBLK, N_HEADS), lambda i, j: (i, 0)),
        ],
        out_specs=pl.BlockSpec((SBLK, TBLK), lambda i, j: (i, j)),
        out_shape=jax.ShapeDtypeStruct((s, s), jnp.float32),
    )(qf, k, w)


def _q_kernel(ql_ref, wq_ref, cos_ref, sin_ref, out_ref):
    x = jnp.dot(ql_ref[...], wq_ref[...], preferred_element_type=jnp.float32)
    lane128 = jax.lax.broadcasted_iota(jnp.int32, (SBLK, HEAD_DIM), 1)
    c = cos_ref[...]
    s_ = sin_ref[...]
    fw_scale = HEAD_DIM ** -0.5
    outs = []
    for hh in range(8):
        xh = x[:, hh * HEAD_DIM : (hh + 1) * HEAD_DIM]
        xr = xh[:, : ROPE_DIM // 2]
        xi = xh[:, ROPE_DIM // 2 : ROPE_DIM]
        rot = jnp.concatenate([xr * c - xi * s_, xr * s_ + xi * c], axis=1)
        y = jnp.concatenate([rot, xh[:, ROPE_DIM:]], axis=1)
        for st in range(7):
            hd = 1 << st
            left = jnp.concatenate([y[:, hd:], y[:, :hd]], axis=1)
            right = jnp.concatenate([y[:, HEAD_DIM - hd :], y[:, : HEAD_DIM - hd]], axis=1)
            y = jnp.where((lane128 & hd) == 0, y + left, right - y)
        outs.append(y * fw_scale)
    out_ref[...] = jnp.concatenate(outs, axis=1)


def _q_proj(ql, wq_b, cos, sin, s):
    nblk = 1024
    return pl.pallas_call(
        _q_kernel,
        grid=(N_HEADS * HEAD_DIM // nblk, s // SBLK),
        in_specs=[
            pl.BlockSpec((SBLK, Q_LORA_RANK), lambda n, i: (i, 0)),
            pl.BlockSpec((Q_LORA_RANK, nblk), lambda n, i: (0, n)),
            pl.BlockSpec((SBLK, ROPE_DIM // 2), lambda n, i: (i, 0)),
            pl.BlockSpec((SBLK, ROPE_DIM // 2), lambda n, i: (i, 0)),
        ],
        out_specs=pl.BlockSpec((SBLK, nblk), lambda n, i: (i, n)),
        out_shape=jax.ShapeDtypeStruct((s, N_HEADS * HEAD_DIM), jnp.float32),
    )(ql, wq_b, cos, sin)


def _topk_sc(masked, s):
    NW = 32
    NB = 256
    mesh = plsc.VectorSubcoreMesh(core_axis_name="c", subcore_axis_name="s")

    def vmem_pair(shape, dt):
        return [pltpu.VMEM(shape, dt), pltpu.VMEM(shape, dt)]

    @functools.partial(
        pl.kernel,
        mesh=mesh,
        compiler_params=pltpu.CompilerParams(needs_layout_passes=False),
        out_type=[
            jax.ShapeDtypeStruct((s, TOPK), jnp.float32),
            jax.ShapeDtypeStruct((s, TOPK), jnp.int32),
        ],
        scratch_types=(
            vmem_pair((s,), jnp.float32)
            + vmem_pair((s,), jnp.int32)
            + vmem_pair((s,), jnp.int32)
            + vmem_pair((s,), jnp.int32)
            + vmem_pair((s,), jnp.int32)
            + vmem_pair((NB * 16,), jnp.int32)
            + vmem_pair((NB,), jnp.int32)
            + vmem_pair((16,), jnp.int32)
            + vmem_pair((TOPK,), jnp.float32)
            + vmem_pair((TOPK,), jnp.int32)
            + [pltpu.SemaphoreType.DMA, pltpu.SemaphoreType.DMA]
        ),
    )
    def tk(scores_hbm, vals_hbm, idx_hbm,
           row_f0, row_f1, keyA0, keyA1, valA0, valA1, keyB0, keyB1,
           valB0, valB1, hist0, hist1, cnt0, cnt1, tmp0, tmp1,
           ovals0, ovals1, oidx0, oidx1, sem0, sem1):
        wid = lax.axis_index("s") * 2 + lax.axis_index("c")
        lane = lax.iota(jnp.int32, 16)
        ones16 = jnp.ones((16,), jnp.int32)
        R = (
            (row_f0, keyA0, valA0, keyB0, valB0, hist0, cnt0, tmp0, ovals0, oidx0),
            (row_f1, keyA1, valA1, keyB1, valB1, hist1, cnt1, tmp1, ovals1, oidx1),
        )

        def hist_rmw(hist, dd):
            slot = dd * 16 + lane
            h0 = plsc.load_gather(hist, [slot])
            plsc.store_scatter(hist, [slot], h0 + ones16)

        def pair_body(g, _):
            row0 = (2 * g) * NW + wid
            row1 = (2 * g + 1) * NW + wid
            rows = (row0, row1)
            cp0 = pltpu.make_async_copy(scores_hbm.at[row0], row_f0, sem0)
            cp1 = pltpu.make_async_copy(scores_hbm.at[row1], row_f1, sem1)
            cp0.start()
            cp1.start()
            cp0.wait()
            cp1.wait()
            nv = row1 // 16 + 1

            def clear_hists(i, _c):
                off = pl.multiple_of(i * 16, 16)
                z = jnp.zeros((16,), jnp.int32)
                for r in range(2):
                    for u in range(16):
                        R[r][5][pl.ds(off * 16 + u * 16, 16)] = z
                return 0

            lax.fori_loop(0, NB // 16, clear_hists, 0)

            def kv_body(i, _c):
                off = pl.multiple_of(i * 16, 16)
                for r in range(2):
                    row_f, keyA, valA, hist = R[r][0], R[r][1], R[r][2], R[r][5]
                    f = row_f[pl.ds(off, 16)]
                    u = lax.bitcast_convert_type(f, jnp.int32)
                    m = jnp.bitwise_not(lax.shift_right_arithmetic(u, 31)) & 0x7FFFFFFF
                    key = u ^ m
                    keyA[pl.ds(off, 16)] = key
                    valA[pl.ds(off, 16)] = lane + i * 16
                    hist_rmw(hist, key & (NB - 1))
                return 0

            lax.fori_loop(0, nv, kv_body, 0)

            for p in range(4):
                shift = 8 * p

                def totals(dv, carry, _p=p):
                    dig = dv * 16 + lane
                    out = []
                    for r in range(2):
                        hist, cnt = R[r][5], R[r][6]
                        tot = jnp.zeros((16,), jnp.int32)
                        for l in range(16):
                            tot = tot + plsc.load_gather(hist, [dig * 16 + l])
                        cs = plsc.cumsum(tot)
                        cnt[pl.ds(pl.multiple_of(dv * 16, 16), 16)] = cs - tot + carry[r]
                        out.append(carry[r] + jnp.sum(tot))
                    return tuple(out)

                lax.fori_loop(0, NB // 16, totals, (jnp.int32(0), jnp.int32(0)))

                lax.fori_loop(0, NB // 16, clear_hists, 0)

                def perm(i, _c, _p=p, _shift=shift):
                    off = pl.multiple_of(i * 16, 16)
                    for r in range(2):
                        refs = R[r]
                        hist, cnt, tmp16 = refs[5], refs[6], refs[7]
                        if _p % 2 == 0:
                            ksrc, vsrc, kdst, vdst = refs[1], refs[2], refs[3], refs[4]
                        else:
                            ksrc, vsrc, kdst, vdst = refs[3], refs[4], refs[1], refs[2]
                        kk = ksrc[pl.ds(off, 16)]
                        dd = lax.shift_right_logical(kk, _shift) & (NB - 1)
                        sdl, slane = plsc.sort_key_val(dd * 16 + lane, lane)
                        sd = lax.shift_right_logical(sdl, 4)
                        tmp16[...] = sd
                        prev = plsc.load_gather(tmp16, [jnp.maximum(lane - 1, 0)])
                        nxt = plsc.load_gather(tmp16, [jnp.minimum(lane + 1, 15)])
                        is_start = (lane == 0) | (prev != sd)
                        is_end = (lane == 15) | (nxt != sd)
                        run_start = plsc.cummax(jnp.where(is_start, lane, 0))
                        rank = lane - run_start
                        base = plsc.load_gather(cnt, [sd])
                        pos = base + rank
                        gidx = i * 16 + slane
                        sk = plsc.load_gather(ksrc, [gidx])
                        sv = plsc.load_gather(vsrc, [gidx])
                        plsc.store_scatter(kdst, [pos], sk)
                        plsc.store_scatter(vdst, [pos], sv)
                        plsc.store_scatter(cnt, [sd], base + rank + 1, mask=is_end)
                        if _p < 3:
                            hist_rmw(hist, lax.shift_right_logical(kk, _shift + 8) & (NB - 1))
                    return 0

                lax.fori_loop(0, nv, perm, 0)

            nout = jnp.minimum(nv, TOPK // 16)

            def out_body(j, _c):
                off = pl.multiple_of(j * 16, 16)
                for r in range(2):
                    row_f, valA, ovals, oidx = R[r][0], R[r][2], R[r][8], R[r][9]
                    iv = valA[pl.ds(off, 16)]
                    ovals[pl.ds(off, 16)] = plsc.load_gather(row_f, [iv])
                    oidx[pl.ds(off, 16)] = iv
                return 0

            lax.fori_loop(0, nout, out_body, 0)

            def fill_body(j, _c):
                off = pl.multiple_of(j * 16, 16)
                for r in range(2):
                    ovals, oidx = R[r][8], R[r][9]
                    oidx[pl.ds(off, 16)] = lane + j * 16
                    ovals[pl.ds(off, 16)] = jnp.full((16,), NEG, jnp.float32)
                return 0

            lax.fori_loop(nout, TOPK // 16, fill_body, 0)

            for r in range(2):
                pltpu.sync_copy(R[r][8], vals_hbm.at[rows[r]])
                pltpu.sync_copy(R[r][9], idx_hbm.at[rows[r]])
            return 0

        lax.fori_loop(0, s // NW // 2, pair_body, 0)

    return tk(masked)


def kernel(hidden_states, q_lora, freqs_cos, freqs_sin, wq_b, wk, k_norm_w, k_norm_b, w_proj):
    b, s, d = hidden_states.shape
    k = hidden_states @ wk
    k = _layer_norm(k, k_norm_w, k_norm_b)
    k_pe = _apply_rope(k[..., :ROPE_DIM][:, :, None, :], freqs_cos, freqs_sin)[:, :, 0, :]
    k = jnp.concatenate([k_pe, k[..., ROPE_DIM:]], axis=-1)
    k = _fwht(k)
    weights = (hidden_states @ w_proj) * (N_HEADS ** -0.5)

    qf = _q_proj(q_lora[0], wq_b, freqs_cos, freqs_sin, s)
    masked = _scores(qf, k[0], weights[0], s)

    topk_vals, topk_idx = _topk_sc(masked, s)
    return topk_vals[None], topk_idx[None]
